# Initial kernel scaffold; baseline (speedup 1.0000x reference)
import jax
import jax.numpy as jnp
from jax.experimental import pallas as pl


def kernel(prototypes):
    raise NotImplementedError("write your pallas kernel here")


# speedup vs baseline: 1.0107x; 1.0107x over previous
import jax
import jax.numpy as jnp
from jax.experimental import pallas as pl

K = 8192
D = 256
_ROWS_PER_BLOCK = 1024


def _normalize_body(x_ref, o_ref):
    x = x_ref[...]
    s = jnp.sum(x * x, axis=1, keepdims=True)
    o_ref[...] = x / jnp.maximum(jnp.sqrt(s), 1e-12)


def kernel(prototypes):
    return pl.pallas_call(
        _normalize_body,
        grid=(K // _ROWS_PER_BLOCK,),
        in_specs=[pl.BlockSpec((_ROWS_PER_BLOCK, D), lambda i: (i, 0))],
        out_specs=pl.BlockSpec((_ROWS_PER_BLOCK, D), lambda i: (i, 0)),
        out_shape=jax.ShapeDtypeStruct((K, D), jnp.float32),
    )(prototypes)


# speedup vs baseline: 1.2697x; 1.2562x over previous
import jax
import jax.numpy as jnp
from jax.experimental import pallas as pl

K = 8192
D = 256
_ROWS_PER_BLOCK = 2048


def _normalize_body(x_ref, o_ref):
    x = x_ref[...]
    s = jnp.sum(x * x, axis=1, keepdims=True)
    o_ref[...] = x / jnp.maximum(jnp.sqrt(s), 1e-12)


def kernel(prototypes):
    return pl.pallas_call(
        _normalize_body,
        grid=(K // _ROWS_PER_BLOCK,),
        in_specs=[pl.BlockSpec((_ROWS_PER_BLOCK, D), lambda i: (i, 0))],
        out_specs=pl.BlockSpec((_ROWS_PER_BLOCK, D), lambda i: (i, 0)),
        out_shape=jax.ShapeDtypeStruct((K, D), jnp.float32),
    )(prototypes)


# speedup vs baseline: 1.5146x; 1.1929x over previous
import jax
import jax.numpy as jnp
from jax.experimental import pallas as pl

K = 8192
D = 256
_ROWS_PER_BLOCK = 4096


def _normalize_body(x_ref, o_ref):
    x = x_ref[...]
    s = jnp.sum(x * x, axis=1, keepdims=True)
    o_ref[...] = x / jnp.maximum(jnp.sqrt(s), 1e-12)


def kernel(prototypes):
    return pl.pallas_call(
        _normalize_body,
        grid=(K // _ROWS_PER_BLOCK,),
        in_specs=[pl.BlockSpec((_ROWS_PER_BLOCK, D), lambda i: (i, 0))],
        out_specs=pl.BlockSpec((_ROWS_PER_BLOCK, D), lambda i: (i, 0)),
        out_shape=jax.ShapeDtypeStruct((K, D), jnp.float32),
    )(prototypes)
